# Initial kernel scaffold; baseline (speedup 1.0000x reference)
import jax
import jax.numpy as jnp
from jax.experimental import pallas as pl


def kernel(gate_logits):
    raise NotImplementedError("write your pallas kernel here")


# speedup vs baseline: 2.3995x; 2.3995x over previous
import functools

import jax
import jax.numpy as jnp
from jax.experimental import pallas as pl
from jax.experimental.pallas import tpu as pltpu

_NUM_EXPERTS = 64
_TOP_K = 8
_ALPHA = 0.01


def _body(x_ref, loss_ref, acc_ref, *, nblocks, total_tokens):
    i = pl.program_id(0)

    @pl.when(i == 0)
    def _init():
        acc_ref[...] = jnp.zeros_like(acc_ref)

    x = x_ref[...]

    m = jnp.max(x, axis=1, keepdims=True)
    e = jnp.exp(x - m)
    s = jnp.sum(e, axis=1, keepdims=True)
    probs = e / s
    psum = jnp.sum(probs, axis=0)

    cur = x
    for _ in range(_TOP_K - 1):
        mk = jnp.max(cur, axis=1, keepdims=True)
        cur = jnp.where(cur == mk, -jnp.inf, cur)
    t8 = jnp.max(cur, axis=1, keepdims=True)
    mask = (x >= t8).astype(jnp.float32)
    fsum = jnp.sum(mask, axis=0)

    acc_ref[0, : _NUM_EXPERTS] += psum
    acc_ref[1, : _NUM_EXPERTS] += fsum

    @pl.when(i == nblocks - 1)
    def _finish():
        p_i = acc_ref[0, : _NUM_EXPERTS] / total_tokens
        f_i = acc_ref[1, : _NUM_EXPERTS] / (total_tokens * _TOP_K)
        loss = _ALPHA * _NUM_EXPERTS * jnp.sum(f_i * p_i)
        loss_ref[...] = jnp.full((1, 1), loss, jnp.float32)


def kernel(gate_logits):
    x = gate_logits.reshape(-1, _NUM_EXPERTS)
    total = x.shape[0]
    block = 2048
    nblocks = total // block
    loss = pl.pallas_call(
        functools.partial(_body, nblocks=nblocks, total_tokens=float(total)),
        grid=(nblocks,),
        in_specs=[pl.BlockSpec((block, _NUM_EXPERTS), lambda i: (i, 0))],
        out_specs=pl.BlockSpec((1, 1), lambda i: (0, 0)),
        out_shape=jax.ShapeDtypeStruct((1, 1), jnp.float32),
        scratch_shapes=[pltpu.VMEM((8, _NUM_EXPERTS), jnp.float32)],
    )(x)
    return loss[0, 0]


# speedup vs baseline: 3.2017x; 1.3343x over previous
import functools

import jax
import jax.numpy as jnp
from jax.experimental import pallas as pl
from jax.experimental.pallas import tpu as pltpu

_NUM_EXPERTS = 64
_TOP_K = 8
_ALPHA = 0.01
_LANES = 128


def _body(x_ref, loss_ref, pacc_ref, facc_ref, *, nblocks, total_tokens):
    i = pl.program_id(0)

    @pl.when(i == 0)
    def _init():
        pacc_ref[...] = jnp.zeros_like(pacc_ref)
        facc_ref[...] = jnp.zeros_like(facc_ref)

    xt = x_ref[...].T
    t = xt.shape[1]

    m1 = jnp.max(xt, axis=0, keepdims=True)
    cur = xt
    mk = m1
    for _ in range(_TOP_K - 1):
        cur = jnp.where(cur >= mk, -jnp.inf, cur)
        mk = jnp.max(cur, axis=0, keepdims=True)
    t8 = mk

    e = jnp.exp(xt - m1)
    s = jnp.sum(e, axis=0, keepdims=True)
    p = e / s
    mask = (xt >= t8).astype(jnp.float32)

    for j in range(t // _LANES):
        pacc_ref[...] += p[:, j * _LANES : (j + 1) * _LANES]
        facc_ref[...] += mask[:, j * _LANES : (j + 1) * _LANES]

    @pl.when(i == nblocks - 1)
    def _finish():
        p_i = jnp.sum(pacc_ref[...], axis=1) / total_tokens
        f_i = jnp.sum(facc_ref[...], axis=1) / (total_tokens * _TOP_K)
        loss = _ALPHA * _NUM_EXPERTS * jnp.sum(f_i * p_i)
        loss_ref[...] = jnp.full((1, 1), loss, jnp.float32)


def kernel(gate_logits):
    x = gate_logits.reshape(-1, _NUM_EXPERTS)
    total = x.shape[0]
    block = 1024
    nblocks = total // block
    loss = pl.pallas_call(
        functools.partial(_body, nblocks=nblocks, total_tokens=float(total)),
        grid=(nblocks,),
        in_specs=[pl.BlockSpec((block, _NUM_EXPERTS), lambda i: (i, 0))],
        out_specs=pl.BlockSpec((1, 1), lambda i: (0, 0)),
        out_shape=jax.ShapeDtypeStruct((1, 1), jnp.float32),
        scratch_shapes=[
            pltpu.VMEM((_NUM_EXPERTS, _LANES), jnp.float32),
            pltpu.VMEM((_NUM_EXPERTS, _LANES), jnp.float32),
        ],
    )(x)
    return loss[0, 0]


# speedup vs baseline: 3.2121x; 1.0033x over previous
import functools

import jax
import jax.numpy as jnp
from jax.experimental import pallas as pl
from jax.experimental.pallas import tpu as pltpu

_NUM_EXPERTS = 64
_TOP_K = 8
_ALPHA = 0.01
_LANES = 128


def _body(x_ref, loss_ref, pacc_ref, facc_ref, *, nblocks, total_tokens):
    i = pl.program_id(0)

    @pl.when(i == 0)
    def _init():
        pacc_ref[...] = jnp.zeros_like(pacc_ref)
        facc_ref[...] = jnp.zeros_like(facc_ref)

    for j in range(x_ref.shape[0] // _LANES):
        xt = x_ref[j * _LANES : (j + 1) * _LANES, :].T

        m1 = jnp.max(xt, axis=0, keepdims=True)
        cur = xt
        mk = m1
        for _ in range(_TOP_K - 1):
            cur = jnp.where(cur >= mk, -jnp.inf, cur)
            mk = jnp.max(cur, axis=0, keepdims=True)
        t8 = mk

        e = jnp.exp(xt - m1)
        s = jnp.sum(e, axis=0, keepdims=True)
        p = e / s
        mask = (xt >= t8).astype(jnp.float32)

        pacc_ref[...] += p
        facc_ref[...] += mask

    @pl.when(i == nblocks - 1)
    def _finish():
        p_i = jnp.sum(pacc_ref[...], axis=1) / total_tokens
        f_i = jnp.sum(facc_ref[...], axis=1) / (total_tokens * _TOP_K)
        loss = _ALPHA * _NUM_EXPERTS * jnp.sum(f_i * p_i)
        loss_ref[...] = jnp.full((1, 1), loss, jnp.float32)


def kernel(gate_logits):
    x = gate_logits.reshape(-1, _NUM_EXPERTS)
    total = x.shape[0]
    block = 1024
    nblocks = total // block
    loss = pl.pallas_call(
        functools.partial(_body, nblocks=nblocks, total_tokens=float(total)),
        grid=(nblocks,),
        in_specs=[pl.BlockSpec((block, _NUM_EXPERTS), lambda i: (i, 0))],
        out_specs=pl.BlockSpec((1, 1), lambda i: (0, 0)),
        out_shape=jax.ShapeDtypeStruct((1, 1), jnp.float32),
        scratch_shapes=[
            pltpu.VMEM((_NUM_EXPERTS, _LANES), jnp.float32),
            pltpu.VMEM((_NUM_EXPERTS, _LANES), jnp.float32),
        ],
    )(x)
    return loss[0, 0]


# speedup vs baseline: 4.3445x; 1.3525x over previous
import functools

import jax
import jax.numpy as jnp
from jax.experimental import pallas as pl
from jax.experimental.pallas import tpu as pltpu

_NUM_EXPERTS = 64
_TOP_K = 8
_ALPHA = 0.01
_LANES = 128


def _body(x_ref, loss_ref, pacc_ref, facc_ref, *, grid_b, grid_t, total_tokens):
    b = pl.program_id(0)
    t = pl.program_id(1)

    @pl.when(jnp.logical_and(b == 0, t == 0))
    def _init():
        pacc_ref[...] = jnp.zeros_like(pacc_ref)
        facc_ref[...] = jnp.zeros_like(facc_ref)

    block = x_ref.shape[1]
    for j in range(block // _LANES):
        xt = x_ref[0, j * _LANES : (j + 1) * _LANES, :].T

        m1 = jnp.max(xt, axis=0, keepdims=True)
        cur = xt
        mk = m1
        for _ in range(_TOP_K - 1):
            cur = jnp.where(cur >= mk, -jnp.inf, cur)
            mk = jnp.max(cur, axis=0, keepdims=True)
        t8 = mk

        e = jnp.exp(xt - m1)
        s = jnp.sum(e, axis=0, keepdims=True)
        p = e / s
        mask = (xt >= t8).astype(jnp.float32)

        pacc_ref[...] += p
        facc_ref[...] += mask

    @pl.when(jnp.logical_and(b == grid_b - 1, t == grid_t - 1))
    def _finish():
        p_i = jnp.sum(pacc_ref[...], axis=1) / total_tokens
        f_i = jnp.sum(facc_ref[...], axis=1) / (total_tokens * _TOP_K)
        loss = _ALPHA * _NUM_EXPERTS * jnp.sum(f_i * p_i)
        loss_ref[...] = jnp.full((1, 1), loss, jnp.float32)


def kernel(gate_logits):
    nb, nt, ne = gate_logits.shape
    total = nb * nt
    block = 1024
    grid_b, grid_t = nb, nt // block
    loss = pl.pallas_call(
        functools.partial(
            _body, grid_b=grid_b, grid_t=grid_t, total_tokens=float(total)
        ),
        grid=(grid_b, grid_t),
        in_specs=[pl.BlockSpec((1, block, ne), lambda i, j: (i, j, 0))],
        out_specs=pl.BlockSpec((1, 1), lambda i, j: (0, 0)),
        out_shape=jax.ShapeDtypeStruct((1, 1), jnp.float32),
        scratch_shapes=[
            pltpu.VMEM((_NUM_EXPERTS, _LANES), jnp.float32),
            pltpu.VMEM((_NUM_EXPERTS, _LANES), jnp.float32),
        ],
    )(gate_logits)
    return loss[0, 0]


# speedup vs baseline: 4.4992x; 1.0356x over previous
import functools

import jax
import jax.numpy as jnp
from jax.experimental import pallas as pl
from jax.experimental.pallas import tpu as pltpu

_NUM_EXPERTS = 64
_TOP_K = 8
_ALPHA = 0.01
_LANES = 128


def _body(x_ref, loss_ref, pacc_ref, facc_ref, *, grid_b, grid_t, total_tokens):
    b = pl.program_id(0)
    t = pl.program_id(1)

    @pl.when(jnp.logical_and(b == 0, t == 0))
    def _init():
        pacc_ref[...] = jnp.zeros_like(pacc_ref)
        facc_ref[...] = jnp.zeros_like(facc_ref)

    _NET = [(0, 1), (2, 3), (4, 5), (6, 7), (0, 2), (1, 3), (4, 6), (5, 7),
            (1, 2), (5, 6), (0, 4), (3, 7), (1, 5), (2, 6), (1, 4), (3, 6),
            (2, 4), (3, 5), (3, 4)]

    block = x_ref.shape[1]
    for j in range(block // _LANES):
        xt = x_ref[0, j * _LANES : (j + 1) * _LANES, :].T

        s8 = [xt[8 * i : 8 * i + 8, :] for i in range(8)]
        for ca, cb in _NET:
            hi = jnp.maximum(s8[ca], s8[cb])
            lo = jnp.minimum(s8[ca], s8[cb])
            s8[ca], s8[cb] = hi, lo

        m1 = None
        for _ in range(_TOP_K - 1):
            g = jnp.max(s8[0], axis=0, keepdims=True)
            if m1 is None:
                m1 = g
            hit = s8[0] >= g
            for i in range(7):
                s8[i] = jnp.where(hit, s8[i + 1], s8[i])
            s8[7] = jnp.where(hit, -jnp.inf, s8[7])
        t8 = jnp.max(s8[0], axis=0, keepdims=True)

        e = jnp.exp(xt - m1)
        s = jnp.sum(e, axis=0, keepdims=True)
        p = e / s
        mask = (xt >= t8).astype(jnp.float32)

        pacc_ref[...] += p
        facc_ref[...] += mask

    @pl.when(jnp.logical_and(b == grid_b - 1, t == grid_t - 1))
    def _finish():
        p_i = jnp.sum(pacc_ref[...], axis=1) / total_tokens
        f_i = jnp.sum(facc_ref[...], axis=1) / (total_tokens * _TOP_K)
        loss = _ALPHA * _NUM_EXPERTS * jnp.sum(f_i * p_i)
        loss_ref[...] = jnp.full((1, 1), loss, jnp.float32)


def kernel(gate_logits):
    nb, nt, ne = gate_logits.shape
    total = nb * nt
    block = 1024
    grid_b, grid_t = nb, nt // block
    loss = pl.pallas_call(
        functools.partial(
            _body, grid_b=grid_b, grid_t=grid_t, total_tokens=float(total)
        ),
        grid=(grid_b, grid_t),
        in_specs=[pl.BlockSpec((1, block, ne), lambda i, j: (i, j, 0))],
        out_specs=pl.BlockSpec((1, 1), lambda i, j: (0, 0)),
        out_shape=jax.ShapeDtypeStruct((1, 1), jnp.float32),
        scratch_shapes=[
            pltpu.VMEM((_NUM_EXPERTS, _LANES), jnp.float32),
            pltpu.VMEM((_NUM_EXPERTS, _LANES), jnp.float32),
        ],
    )(gate_logits)
    return loss[0, 0]
